# baseline (speedup 1.0000x reference)
import jax
import jax.numpy as jnp
from jax.experimental import pallas as pl


def _isin_sorted(queries, table):
    t = jnp.sort(table)
    idx = jnp.clip(jnp.searchsorted(t, queries), 0, t.shape[0] - 1)
    return t[idx] == queries


def _gcn(x, W, b, s, d, w, N):
    h = x @ W
    deg = jax.ops.segment_sum(w, d, num_segments=N)
    dinv = jnp.where(deg > 0, 1.0 / jnp.sqrt(jnp.maximum(deg, 1e-12)), 0.0)
    norm = dinv[s] * dinv[d] * w
    out = jax.ops.segment_sum(h[s] * norm[:, None], d, num_segments=N)
    return out + b


def _gcn_dense(x, W, b, M, N):
    h = x @ W
    Mf = M.astype(x.dtype)
    deg = jnp.sum(Mf, axis=0)
    dinv = jnp.where(deg > 0, 1.0 / jnp.sqrt(jnp.maximum(deg, 1e-12)), 0.0)
    Wm = Mf * (dinv[:, None] * dinv[None, :])
    out = jnp.matmul(Wm.T, h, precision=jax.lax.Precision.HIGHEST)
    return out + b


def _ident_kernel(x_ref, o_ref):
    o_ref[...] = x_ref[...]


def kernel(x, edge_index, W1, b1, W2, b2, Wl, bl):
    N = x.shape[0]
    src0 = edge_index[0].astype(jnp.int32)
    dst0 = edge_index[1].astype(jnp.int32)
    keys_orig = src0 * N + dst0
    loops = jnp.arange(N, dtype=jnp.int32)
    t_src = jnp.concatenate([src0, loops])
    t_dst = jnp.concatenate([dst0, loops])
    keep = ~_isin_sorted(t_src * N + t_dst, keys_orig)
    m = src0 != dst0
    A = jnp.zeros((N, N), x.dtype).at[src0, dst0].set(1.0)
    A_E = A.at[loops, loops].set(0.0)
    A_loop = A.at[loops, loops].set(1.0)
    P = jnp.matmul(A_loop, A_loop, precision=jax.lax.Precision.HIGHEST)
    M2 = (P > 0) & (A_E == 0)
    M2 = M2.at[loops, loops].set(False)
    xn = x / (jnp.linalg.norm(x, axis=1, keepdims=True) + 1e-12)
    sim = xn @ xn.T
    sim = sim.at[jnp.arange(N), jnp.arange(N)].set(-jnp.inf)
    _, nbr = jax.lax.top_k(sim, 5)
    knn_src = jnp.repeat(jnp.arange(N, dtype=jnp.int32), 5)
    knn_dst = nbr.reshape(-1).astype(jnp.int32)
    s1, d1, w1v = src0, dst0, m.astype(x.dtype)
    s3, d3, w3v = knn_src, knn_dst, jnp.ones(knn_src.shape[0], x.dtype)
    s4, d4, w4v = t_src, t_dst, keep.astype(x.dtype)

    h1 = [
        _gcn(x, W1, b1, s1, d1, w1v, N),
        _gcn_dense(x, W1, b1, M2, N),
        _gcn(x, W1, b1, s3, d3, w3v, N),
        _gcn(x, W1, b1, s4, d4, w4v, N),
    ]
    R1 = jax.nn.relu(jnp.concatenate(h1, axis=1))
    h2 = [
        _gcn(R1, W2, b2, s1, d1, w1v, N),
        _gcn_dense(R1, W2, b2, M2, N),
        _gcn(R1, W2, b2, s3, d3, w3v, N),
        _gcn(R1, W2, b2, s4, d4, w4v, N),
    ]
    R2 = jnp.concatenate(h2, axis=1)
    out = jax.nn.log_softmax(R2 @ Wl + bl, axis=1)
    out = pl.pallas_call(
        _ident_kernel,
        out_shape=jax.ShapeDtypeStruct(out.shape, out.dtype),
    )(out)
    return out


# speedup vs baseline: 4.2045x; 4.2045x over previous
import functools

import jax
import jax.numpy as jnp
from jax.experimental import pallas as pl
from jax.experimental.pallas import tpu as pltpu

N = 10000
E = 160000
Nd = 10240
BM = 1024
KT = Nd // BM
RB = 256
RL = 512

_NEG = -1e30


def _dinv(deg):
    return jnp.where(deg > 0, jax.lax.rsqrt(jnp.maximum(deg, 1e-12)), 0.0)


def _xn_body(x_ref, o_ref):
    x = x_ref[...]
    s = jnp.sum(x * x, axis=1, keepdims=True)
    o_ref[...] = x / (jnp.sqrt(s) + 1e-12)


def _xn(x_pad):
    return pl.pallas_call(
        _xn_body,
        grid=(Nd // RL,),
        in_specs=[pl.BlockSpec((RL, 128), lambda i: (i, 0))],
        out_specs=pl.BlockSpec((RL, 128), lambda i: (i, 0)),
        out_shape=jax.ShapeDtypeStruct((Nd, 128), jnp.float32),
    )(x_pad)


def _knn_body(xn_ref, xnt_ref, o_ref):
    i = pl.program_id(0)
    a = xn_ref[...]
    b = xnt_ref[...]
    sim = jnp.dot(a, b, preferred_element_type=jnp.float32)
    rows = jax.lax.broadcasted_iota(jnp.int32, (RB, Nd), 0) + i * RB
    cols = jax.lax.broadcasted_iota(jnp.int32, (RB, Nd), 1)
    sim = jnp.where((cols == rows) | (cols >= N), _NEG, sim)
    idxs = []
    for _ in range(5):
        m = jnp.max(sim, axis=1, keepdims=True)
        isel = jnp.min(jnp.where(sim >= m, cols, jnp.int32(1 << 30)),
                       axis=1, keepdims=True)
        idxs.append(isel)
        sim = jnp.where(cols == isel, _NEG, sim)
    nbr = jnp.concatenate(idxs + [jnp.zeros((RB, 3), jnp.int32)], axis=1)
    o_ref[...] = nbr


def _knn(xn, xnt):
    return pl.pallas_call(
        _knn_body,
        grid=(Nd // RB,),
        in_specs=[
            pl.BlockSpec((RB, 128), lambda i: (i, 0)),
            pl.BlockSpec((128, Nd), lambda i: (0, 0)),
        ],
        out_specs=pl.BlockSpec((RB, 8), lambda i: (i, 0)),
        out_shape=jax.ShapeDtypeStruct((Nd, 8), jnp.int32),
    )(xn, xnt)


def _m2_body(a_ik, a_kj, a_ij, m2_ref, deg_ref, acc_ref, dacc_ref):
    i, j, k = pl.program_id(0), pl.program_id(1), pl.program_id(2)

    @pl.when(k == 0)
    def _():
        acc_ref[...] = jnp.zeros_like(acc_ref)

    acc_ref[...] += jnp.dot(a_ik[...], a_kj[...],
                            preferred_element_type=jnp.float32)

    @pl.when(k == KT - 1)
    def _():
        p = acc_ref[...]
        rows = jax.lax.broadcasted_iota(jnp.int32, (BM, BM), 0) + i * BM
        cols = jax.lax.broadcasted_iota(jnp.int32, (BM, BM), 1) + j * BM
        m2 = (p > 0) & (a_ij[...] == 0) & (rows != cols)
        m2f = m2.astype(jnp.float32)
        m2_ref[...] = m2f.astype(jnp.bfloat16)
        part = jnp.dot(m2f, jnp.ones((BM, 64), jnp.float32),
                       preferred_element_type=jnp.float32)

        @pl.when(j == 0)
        def _():
            dacc_ref[...] = jnp.zeros_like(dacc_ref)

        dacc_ref[...] += part

        @pl.when(j == KT - 1)
        def _():
            deg_ref[...] = dacc_ref[...]


def _m2(atl):
    return pl.pallas_call(
        _m2_body,
        grid=(KT, KT, KT),
        in_specs=[
            pl.BlockSpec((BM, BM), lambda i, j, k: (i, k)),
            pl.BlockSpec((BM, BM), lambda i, j, k: (k, j)),
            pl.BlockSpec((BM, BM), lambda i, j, k: (i, j)),
        ],
        out_specs=[
            pl.BlockSpec((BM, BM), lambda i, j, k: (i, j)),
            pl.BlockSpec((BM, 64), lambda i, j, k: (i, 0)),
        ],
        out_shape=[
            jax.ShapeDtypeStruct((Nd, Nd), jnp.bfloat16),
            jax.ShapeDtypeStruct((Nd, 64), jnp.float32),
        ],
        scratch_shapes=[
            pltpu.VMEM((BM, BM), jnp.float32),
            pltpu.VMEM((BM, 64), jnp.float32),
        ],
        compiler_params=pltpu.CompilerParams(
            dimension_semantics=("parallel", "arbitrary", "arbitrary")),
    )(atl, atl, atl)


def _dgcn_body(m2_ref, h_ref, degk_ref, degi_ref, o_ref, acc_ref):
    k = pl.program_id(1)

    @pl.when(k == 0)
    def _():
        acc_ref[...] = jnp.zeros_like(acc_ref)

    g = _dinv(degk_ref[...]) * h_ref[...]
    m2 = m2_ref[...].astype(jnp.float32)
    acc_ref[...] += jnp.dot(m2, g, preferred_element_type=jnp.float32,
                            precision=jax.lax.Precision.HIGHEST)

    @pl.when(k == KT - 1)
    def _():
        o_ref[...] = _dinv(degi_ref[...]) * acc_ref[...]


def _dgcn(m2t, h, deg2):
    return pl.pallas_call(
        _dgcn_body,
        grid=(KT, KT),
        in_specs=[
            pl.BlockSpec((BM, BM), lambda i, k: (i, k)),
            pl.BlockSpec((BM, 64), lambda i, k: (k, 0)),
            pl.BlockSpec((BM, 64), lambda i, k: (k, 0)),
            pl.BlockSpec((BM, 64), lambda i, k: (i, 0)),
        ],
        out_specs=pl.BlockSpec((BM, 64), lambda i, k: (i, 0)),
        out_shape=jax.ShapeDtypeStruct((Nd, 64), jnp.float32),
        scratch_shapes=[pltpu.VMEM((BM, 64), jnp.float32)],
        compiler_params=pltpu.CompilerParams(
            dimension_semantics=("parallel", "arbitrary")),
    )(m2t, h, deg2, deg2)


def _head1_body(x_ref, w_ref, d1_ref, d3_ref, d2_ref,
                h_ref, hs1_ref, hs3_ref, g_ref):
    h = jnp.dot(x_ref[...], w_ref[...], preferred_element_type=jnp.float32)
    h_ref[...] = h
    hs1_ref[...] = _dinv(d1_ref[...]) * h
    hs3_ref[...] = _dinv(d3_ref[...]) * h
    g_ref[...] = _dinv(d2_ref[...]) * h


def _head1(x_pad, W1, deg1, deg3, deg2):
    o = jax.ShapeDtypeStruct((Nd, 64), jnp.float32)
    return pl.pallas_call(
        _head1_body,
        grid=(Nd // RL,),
        in_specs=[
            pl.BlockSpec((RL, 128), lambda i: (i, 0)),
            pl.BlockSpec((128, 64), lambda i: (0, 0)),
            pl.BlockSpec((RL, 1), lambda i: (i, 0)),
            pl.BlockSpec((RL, 1), lambda i: (i, 0)),
            pl.BlockSpec((RL, 64), lambda i: (i, 0)),
        ],
        out_specs=[pl.BlockSpec((RL, 64), lambda i: (i, 0))] * 4,
        out_shape=[o, o, o, o],
    )(x_pad, W1, deg1, deg3, deg2)


def _combine_body(o1a, o1b, o3a, o3b, ob, h_ref, keep_ref,
                  d1_ref, d3_ref, d2_ref, w_ref, bias_ref,
                  h2_ref, hs1_ref, hs3_ref, g_ref):
    b = bias_ref[...]
    a1 = _dinv(d1_ref[...]) * (o1a[...] + o1b[...]) + b
    a2 = ob[...] + b
    a3 = _dinv(d3_ref[...]) * (o3a[...] + o3b[...]) + b
    a4 = keep_ref[...] * h_ref[...] + b
    r = jax.nn.relu(jnp.concatenate([a1, a2, a3, a4], axis=1))
    h2 = jnp.dot(r, w_ref[...], preferred_element_type=jnp.float32)
    h2_ref[...] = h2
    hs1_ref[...] = _dinv(d1_ref[...]) * h2
    hs3_ref[...] = _dinv(d3_ref[...]) * h2
    g_ref[...] = _dinv(d2_ref[...]) * h2


def _combine(o1a, o1b, o3a, o3b, ob, h, keep, deg1, deg3, deg2, W2, b1r):
    o = jax.ShapeDtypeStruct((Nd, 64), jnp.float32)
    return pl.pallas_call(
        _combine_body,
        grid=(Nd // RL,),
        in_specs=[
            pl.BlockSpec((RL, 64), lambda i: (i, 0)),
            pl.BlockSpec((RL, 64), lambda i: (i, 0)),
            pl.BlockSpec((RL, 64), lambda i: (i, 0)),
            pl.BlockSpec((RL, 64), lambda i: (i, 0)),
            pl.BlockSpec((RL, 64), lambda i: (i, 0)),
            pl.BlockSpec((RL, 64), lambda i: (i, 0)),
            pl.BlockSpec((RL, 1), lambda i: (i, 0)),
            pl.BlockSpec((RL, 1), lambda i: (i, 0)),
            pl.BlockSpec((RL, 1), lambda i: (i, 0)),
            pl.BlockSpec((RL, 64), lambda i: (i, 0)),
            pl.BlockSpec((256, 64), lambda i: (0, 0)),
            pl.BlockSpec((1, 64), lambda i: (0, 0)),
        ],
        out_specs=[pl.BlockSpec((RL, 64), lambda i: (i, 0))] * 4,
        out_shape=[o, o, o, o],
    )(o1a, o1b, o3a, o3b, ob, h, keep, deg1, deg3, deg2, W2, b1r)


def _final_body(o1a, o1b, o3a, o3b, ob, h_ref, keep_ref,
                d1_ref, d3_ref, w_ref, bias_ref, bl_ref, o_ref):
    b = bias_ref[...]
    a1 = _dinv(d1_ref[...]) * (o1a[...] + o1b[...]) + b
    a2 = ob[...] + b
    a3 = _dinv(d3_ref[...]) * (o3a[...] + o3b[...]) + b
    a4 = keep_ref[...] * h_ref[...] + b
    r = jnp.concatenate([a1, a2, a3, a4], axis=1)
    z = jnp.dot(r, w_ref[...], preferred_element_type=jnp.float32) + bl_ref[...]
    m = jnp.max(z, axis=1, keepdims=True)
    lse = m + jnp.log(jnp.sum(jnp.exp(z - m), axis=1, keepdims=True))
    o_ref[...] = z - lse


def _final(o1a, o1b, o3a, o3b, ob, h, keep, deg1, deg3, Wl, b2r, blr):
    return pl.pallas_call(
        _final_body,
        grid=(Nd // RL,),
        in_specs=[
            pl.BlockSpec((RL, 64), lambda i: (i, 0)),
            pl.BlockSpec((RL, 64), lambda i: (i, 0)),
            pl.BlockSpec((RL, 64), lambda i: (i, 0)),
            pl.BlockSpec((RL, 64), lambda i: (i, 0)),
            pl.BlockSpec((RL, 64), lambda i: (i, 0)),
            pl.BlockSpec((RL, 64), lambda i: (i, 0)),
            pl.BlockSpec((RL, 1), lambda i: (i, 0)),
            pl.BlockSpec((RL, 1), lambda i: (i, 0)),
            pl.BlockSpec((RL, 1), lambda i: (i, 0)),
            pl.BlockSpec((256, 64), lambda i: (0, 0)),
            pl.BlockSpec((1, 64), lambda i: (0, 0)),
            pl.BlockSpec((1, 64), lambda i: (0, 0)),
        ],
        out_specs=pl.BlockSpec((RL, 64), lambda i: (i, 0)),
        out_shape=jax.ShapeDtypeStruct((Nd, 64), jnp.float32),
    )(o1a, o1b, o3a, o3b, ob, h, keep, deg1, deg3, Wl, b2r, blr)


def _seg_gcn(h, s, d, w, dinv):
    hs = dinv[:N, 0][:, None] * h[:N]
    out = jax.ops.segment_sum(hs[s] * w[:, None], d, num_segments=N)
    return jnp.pad(out, ((0, Nd - N), (0, 0)))


def kernel(x, edge_index, W1, b1, W2, b2, Wl, bl):
    src0 = edge_index[0].astype(jnp.int32)
    dst0 = edge_index[1].astype(jnp.int32)
    x_pad = jnp.pad(x, ((0, Nd - N), (0, 0)))

    loops = jnp.arange(N, dtype=jnp.int32)
    a_raw = jnp.zeros((Nd, Nd), jnp.bfloat16).at[dst0, src0].set(1.0)
    keep = 1.0 - (jnp.diagonal(a_raw)[:N] != 0).astype(jnp.float32)
    keep = jnp.pad(keep, (0, Nd - N))[:, None]
    atl = a_raw.at[loops, loops].set(1.0)

    xn = _xn(x_pad)
    nbr = _knn(xn, xn.T)
    knn_dst = nbr[:N, :5].reshape(-1)
    knn_src = jnp.repeat(jnp.arange(N, dtype=jnp.int32), 5)

    m2t, deg2 = _m2(atl)

    w1v = (src0 != dst0).astype(jnp.float32)
    deg1 = jax.ops.segment_sum(w1v, dst0, num_segments=N)
    deg1 = jnp.pad(deg1, (0, Nd - N))[:, None]
    deg3 = jax.ops.segment_sum(jnp.ones((5 * N,), jnp.float32), knn_dst,
                               num_segments=N)
    deg3 = jnp.pad(deg3, (0, Nd - N))[:, None]
    dinv1 = _dinv(deg1)
    dinv3 = _dinv(deg3)

    b1r = b1[None, :]
    b2r = b2[None, :]
    blr = bl[None, :]

    h1, hs1, hs3, g1 = _head1(x_pad, W1, deg1, deg3, deg2)
    del hs1, hs3
    o1 = _seg_gcn(h1, src0, dst0, w1v, dinv1)
    o3 = _seg_gcn(h1, knn_src, knn_dst, jnp.ones((5 * N,), jnp.float32), dinv3)
    ob1 = _dgcn(m2t, h1, deg2)
    z = jnp.zeros((Nd, 64), jnp.float32)

    h2, hs1b, hs3b, g2 = _combine(o1, z, o3, z, ob1, h1, keep,
                                  deg1, deg3, deg2, W2, b1r)
    del hs1b, hs3b
    o1b = _seg_gcn(h2, src0, dst0, w1v, dinv1)
    o3b = _seg_gcn(h2, knn_src, knn_dst, jnp.ones((5 * N,), jnp.float32), dinv3)
    ob2 = _dgcn(m2t, h2, deg2)

    out = _final(o1b, z, o3b, z, ob2, h2, keep, deg1, deg3, Wl, b2r, blr)
    return out[:N]


# speedup vs baseline: 4.8506x; 1.1537x over previous
import functools

import jax
import jax.numpy as jnp
from jax import lax
from jax.experimental import pallas as pl
from jax.experimental.pallas import tpu as pltpu
from jax.experimental.pallas import tpu_sc as plsc

N = 10000
E = 160000
Nd = 10240
NC = 2
NS = 16
CH = 128
E1P = 163840
E3P = 53248
BM = 1024
KT = Nd // BM
RB = 256
RL = 512

_NEG = -1e30


def _dinv(deg):
    return jnp.where(deg > 0, jax.lax.rsqrt(jnp.maximum(deg, 1e-12)), 0.0)


def _xn_body(x_ref, o_ref):
    x = x_ref[...]
    s = jnp.sum(x * x, axis=1, keepdims=True)
    o_ref[...] = x / (jnp.sqrt(s) + 1e-12)


def _xn(x_pad):
    return pl.pallas_call(
        _xn_body,
        grid=(Nd // RL,),
        in_specs=[pl.BlockSpec((RL, 128), lambda i: (i, 0))],
        out_specs=pl.BlockSpec((RL, 128), lambda i: (i, 0)),
        out_shape=jax.ShapeDtypeStruct((Nd, 128), jnp.float32),
    )(x_pad)


def _knn_body(xn_ref, xnt_ref, o_ref):
    i = pl.program_id(0)
    a = xn_ref[...]
    b = xnt_ref[...]
    sim = jnp.dot(a, b, preferred_element_type=jnp.float32)
    rows = jax.lax.broadcasted_iota(jnp.int32, (RB, Nd), 0) + i * RB
    cols = jax.lax.broadcasted_iota(jnp.int32, (RB, Nd), 1)
    sim = jnp.where((cols == rows) | (cols >= N), _NEG, sim)
    idxs = []
    for _ in range(5):
        m = jnp.max(sim, axis=1, keepdims=True)
        isel = jnp.min(jnp.where(sim >= m, cols, jnp.int32(1 << 30)),
                       axis=1, keepdims=True)
        idxs.append(isel)
        sim = jnp.where(cols == isel, _NEG, sim)
    nbr = jnp.concatenate(idxs + [jnp.zeros((RB, 3), jnp.int32)], axis=1)
    o_ref[...] = nbr


def _knn(xn, xnt):
    return pl.pallas_call(
        _knn_body,
        grid=(Nd // RB,),
        in_specs=[
            pl.BlockSpec((RB, 128), lambda i: (i, 0)),
            pl.BlockSpec((128, Nd), lambda i: (0, 0)),
        ],
        out_specs=pl.BlockSpec((RB, 8), lambda i: (i, 0)),
        out_shape=jax.ShapeDtypeStruct((Nd, 8), jnp.int32),
    )(xn, xnt)


def _m2_body(a_ik, a_kj, a_ij, m2_ref, deg_ref, acc_ref, dacc_ref):
    i, j, k = pl.program_id(0), pl.program_id(1), pl.program_id(2)

    @pl.when(k == 0)
    def _():
        acc_ref[...] = jnp.zeros_like(acc_ref)

    acc_ref[...] += jnp.dot(a_ik[...], a_kj[...],
                            preferred_element_type=jnp.float32)

    @pl.when(k == KT - 1)
    def _():
        p = acc_ref[...]
        rows = jax.lax.broadcasted_iota(jnp.int32, (BM, BM), 0) + i * BM
        cols = jax.lax.broadcasted_iota(jnp.int32, (BM, BM), 1) + j * BM
        m2 = (p > 0) & (a_ij[...] == 0) & (rows != cols)
        m2f = m2.astype(jnp.float32)
        m2_ref[...] = m2f.astype(jnp.bfloat16)
        part = jnp.dot(m2f, jnp.ones((BM, 64), jnp.float32),
                       preferred_element_type=jnp.float32)

        @pl.when(j == 0)
        def _():
            dacc_ref[...] = jnp.zeros_like(dacc_ref)

        dacc_ref[...] += part

        @pl.when(j == KT - 1)
        def _():
            deg_ref[...] = dacc_ref[...]


def _m2(atl):
    return pl.pallas_call(
        _m2_body,
        grid=(KT, KT, KT),
        in_specs=[
            pl.BlockSpec((BM, BM), lambda i, j, k: (i, k)),
            pl.BlockSpec((BM, BM), lambda i, j, k: (k, j)),
            pl.BlockSpec((BM, BM), lambda i, j, k: (i, j)),
        ],
        out_specs=[
            pl.BlockSpec((BM, BM), lambda i, j, k: (i, j)),
            pl.BlockSpec((BM, 64), lambda i, j, k: (i, 0)),
        ],
        out_shape=[
            jax.ShapeDtypeStruct((Nd, Nd), jnp.bfloat16),
            jax.ShapeDtypeStruct((Nd, 64), jnp.float32),
        ],
        scratch_shapes=[
            pltpu.VMEM((BM, BM), jnp.float32),
            pltpu.VMEM((BM, 64), jnp.float32),
        ],
        compiler_params=pltpu.CompilerParams(
            dimension_semantics=("parallel", "arbitrary", "arbitrary")),
    )(atl, atl, atl)


def _dgcn_body(m2_ref, h_ref, degk_ref, degi_ref, o_ref, acc_ref):
    k = pl.program_id(1)

    @pl.when(k == 0)
    def _():
        acc_ref[...] = jnp.zeros_like(acc_ref)

    g = _dinv(degk_ref[...]) * h_ref[...]
    m2 = m2_ref[...].astype(jnp.float32)
    acc_ref[...] += jnp.dot(m2, g, preferred_element_type=jnp.float32,
                            precision=jax.lax.Precision.HIGHEST)

    @pl.when(k == KT - 1)
    def _():
        o_ref[...] = _dinv(degi_ref[...]) * acc_ref[...]


def _dgcn(m2t, h, deg2):
    return pl.pallas_call(
        _dgcn_body,
        grid=(KT, KT),
        in_specs=[
            pl.BlockSpec((BM, BM), lambda i, k: (i, k)),
            pl.BlockSpec((BM, 64), lambda i, k: (k, 0)),
            pl.BlockSpec((BM, 64), lambda i, k: (k, 0)),
            pl.BlockSpec((BM, 64), lambda i, k: (i, 0)),
        ],
        out_specs=pl.BlockSpec((BM, 64), lambda i, k: (i, 0)),
        out_shape=jax.ShapeDtypeStruct((Nd, 64), jnp.float32),
        scratch_shapes=[pltpu.VMEM((BM, 64), jnp.float32)],
        compiler_params=pltpu.CompilerParams(
            dimension_semantics=("parallel", "arbitrary")),
    )(m2t, h, deg2, deg2)


def _head1_body(x_ref, w_ref, d1_ref, d3_ref, h_ref, hs_ref):
    h = jnp.dot(x_ref[...], w_ref[...], preferred_element_type=jnp.float32)
    h_ref[...] = h
    hs_ref[...] = jnp.concatenate(
        [_dinv(d1_ref[...]) * h, _dinv(d3_ref[...]) * h], axis=1)


def _head1(x_pad, W1, deg1, deg3):
    return pl.pallas_call(
        _head1_body,
        grid=(Nd // RL,),
        in_specs=[
            pl.BlockSpec((RL, 128), lambda i: (i, 0)),
            pl.BlockSpec((128, 64), lambda i: (0, 0)),
            pl.BlockSpec((RL, 1), lambda i: (i, 0)),
            pl.BlockSpec((RL, 1), lambda i: (i, 0)),
        ],
        out_specs=[
            pl.BlockSpec((RL, 64), lambda i: (i, 0)),
            pl.BlockSpec((RL, 128), lambda i: (i, 0)),
        ],
        out_shape=[
            jax.ShapeDtypeStruct((Nd, 64), jnp.float32),
            jax.ShapeDtypeStruct((Nd, 128), jnp.float32),
        ],
    )(x_pad, W1, deg1, deg3)


def _combine_body(o1, o3, ob, h_ref, keep_ref,
                  d1_ref, d3_ref, w_ref, bias_ref, h2_ref, hs_ref):
    b = bias_ref[...]
    a1 = _dinv(d1_ref[...]) * o1[...] + b
    a2 = ob[...] + b
    a3 = _dinv(d3_ref[...]) * o3[...] + b
    a4 = keep_ref[...] * h_ref[...] + b
    r = jax.nn.relu(jnp.concatenate([a1, a2, a3, a4], axis=1))
    h2 = jnp.dot(r, w_ref[...], preferred_element_type=jnp.float32)
    h2_ref[...] = h2
    hs_ref[...] = jnp.concatenate(
        [_dinv(d1_ref[...]) * h2, _dinv(d3_ref[...]) * h2], axis=1)


def _combine(o1, o3, ob, h, keep, deg1, deg3, W2, b1r):
    return pl.pallas_call(
        _combine_body,
        grid=(Nd // RL,),
        in_specs=[
            pl.BlockSpec((RL, 64), lambda i: (i, 0)),
            pl.BlockSpec((RL, 64), lambda i: (i, 0)),
            pl.BlockSpec((RL, 64), lambda i: (i, 0)),
            pl.BlockSpec((RL, 64), lambda i: (i, 0)),
            pl.BlockSpec((RL, 1), lambda i: (i, 0)),
            pl.BlockSpec((RL, 1), lambda i: (i, 0)),
            pl.BlockSpec((RL, 1), lambda i: (i, 0)),
            pl.BlockSpec((256, 64), lambda i: (0, 0)),
            pl.BlockSpec((1, 64), lambda i: (0, 0)),
        ],
        out_specs=[
            pl.BlockSpec((RL, 64), lambda i: (i, 0)),
            pl.BlockSpec((RL, 128), lambda i: (i, 0)),
        ],
        out_shape=[
            jax.ShapeDtypeStruct((Nd, 64), jnp.float32),
            jax.ShapeDtypeStruct((Nd, 128), jnp.float32),
        ],
    )(o1, o3, ob, h, keep, deg1, deg3, W2, b1r)


def _final_body(o1, o3, ob, h_ref, keep_ref,
                d1_ref, d3_ref, w_ref, bias_ref, bl_ref, o_ref):
    b = bias_ref[...]
    a1 = _dinv(d1_ref[...]) * o1[...] + b
    a2 = ob[...] + b
    a3 = _dinv(d3_ref[...]) * o3[...] + b
    a4 = keep_ref[...] * h_ref[...] + b
    r = jnp.concatenate([a1, a2, a3, a4], axis=1)
    z = jnp.dot(r, w_ref[...], preferred_element_type=jnp.float32) + bl_ref[...]
    m = jnp.max(z, axis=1, keepdims=True)
    lse = m + jnp.log(jnp.sum(jnp.exp(z - m), axis=1, keepdims=True))
    o_ref[...] = z - lse


def _final(o1, o3, ob, h, keep, deg1, deg3, Wl, b2r, blr):
    return pl.pallas_call(
        _final_body,
        grid=(Nd // RL,),
        in_specs=[
            pl.BlockSpec((RL, 64), lambda i: (i, 0)),
            pl.BlockSpec((RL, 64), lambda i: (i, 0)),
            pl.BlockSpec((RL, 64), lambda i: (i, 0)),
            pl.BlockSpec((RL, 64), lambda i: (i, 0)),
            pl.BlockSpec((RL, 1), lambda i: (i, 0)),
            pl.BlockSpec((RL, 1), lambda i: (i, 0)),
            pl.BlockSpec((RL, 1), lambda i: (i, 0)),
            pl.BlockSpec((256, 64), lambda i: (0, 0)),
            pl.BlockSpec((1, 64), lambda i: (0, 0)),
            pl.BlockSpec((1, 64), lambda i: (0, 0)),
        ],
        out_specs=pl.BlockSpec((RL, 64), lambda i: (i, 0)),
        out_shape=jax.ShapeDtypeStruct((Nd, 64), jnp.float32),
    )(o1, o3, ob, h, keep, deg1, deg3, Wl, b2r, blr)


def _sc_mesh():
    return plsc.VectorSubcoreMesh(core_axis_name="c", subcore_axis_name="s")


def _zero_stripe(zeros_hbm, shared, lo, rows):
    pltpu.sync_copy(zeros_hbm.at[pl.ds(0, rows)], shared.at[pl.ds(lo, rows)])


def _redirect(svm, dvm):
    for v in range(CH // 16):
        sl = pl.ds(v * 16, 16)
        sv = svm[sl]
        dv = dvm[sl]
        dvm[sl] = jnp.where(sv == dv, jnp.int32(N), dv)


def _sc_deg(s1p, d1p, d3p, ones128, zeros128):
    n1 = E1P // (NS * CH)
    n3 = E3P // (NS * CH)

    @functools.partial(
        pl.kernel,
        out_type=jax.ShapeDtypeStruct((NC, Nd, 128), jnp.float32),
        mesh=_sc_mesh(),
        scratch_types=[
            pltpu.VMEM((CH,), jnp.int32),
            pltpu.VMEM((CH,), jnp.int32),
            pltpu.VMEM((CH, 128), jnp.float32),
            pltpu.VMEM_SHARED((Nd, 128), jnp.float32),
        ],
    )
    def k(s1_hbm, d1_hbm, d3_hbm, ones_hbm, z_hbm, o_hbm,
          svm, dvm, ones_v, sh):
        c = lax.axis_index("c")
        s = lax.axis_index("s")
        stripe = s * (Nd // NS)
        pltpu.sync_copy(ones_hbm, ones_v)
        for t in range(Nd // NS // CH):
            _zero_stripe(z_hbm, sh, stripe + t * CH, CH)
        plsc.subcore_barrier()

        @pl.when(c == 0)
        def _():
            def body1(t, _):
                base = (s * n1 + t) * CH
                pltpu.sync_copy(s1_hbm.at[pl.ds(base, CH)], svm)
                pltpu.sync_copy(d1_hbm.at[pl.ds(base, CH)], dvm)
                _redirect(svm, dvm)
                pltpu.sync_copy(ones_v, sh.at[dvm], add=True)
                return 0

            lax.fori_loop(0, n1, body1, 0)

        @pl.when(c == 1)
        def _():
            def body3(t, _):
                base = (s * n3 + t) * CH
                pltpu.sync_copy(d3_hbm.at[pl.ds(base, CH)], dvm)
                pltpu.sync_copy(ones_v, sh.at[dvm], add=True)
                return 0

            lax.fori_loop(0, n3, body3, 0)

        plsc.subcore_barrier()
        rows = Nd // NS
        pltpu.sync_copy(sh.at[pl.ds(stripe, rows)],
                        o_hbm.at[c, pl.ds(stripe, rows)])

    return k(s1p, d1p, d3p, ones128, zeros128)


def _sc_layer(hspack, s1p, d1p, s3p, d3p, zeros128):
    n1 = E1P // (NS * CH)
    n3 = E3P // (NS * CH)

    @functools.partial(
        pl.kernel,
        out_type=jax.ShapeDtypeStruct((NC, Nd, 128), jnp.float32),
        mesh=_sc_mesh(),
        scratch_types=[
            pltpu.VMEM((CH,), jnp.int32),
            pltpu.VMEM((CH,), jnp.int32),
            pltpu.VMEM((CH, 128), jnp.float32),
            pltpu.VMEM_SHARED((Nd, 128), jnp.float32),
            pltpu.SemaphoreType.DMA,
        ],
    )
    def k(hs_hbm, s1_hbm, d1_hbm, s3_hbm, d3_hbm, z_hbm, o_hbm,
          svm, dvm, rows_v, sh, sem):
        c = lax.axis_index("c")
        s = lax.axis_index("s")
        stripe = s * (Nd // NS)
        for t in range(Nd // NS // CH):
            _zero_stripe(z_hbm, sh, stripe + t * CH, CH)
        plsc.subcore_barrier()

        @pl.when(c == 0)
        def _():
            def body1(t, _):
                base = (s * n1 + t) * CH
                pltpu.sync_copy(s1_hbm.at[pl.ds(base, CH)], svm)
                pltpu.sync_copy(d1_hbm.at[pl.ds(base, CH)], dvm)
                _redirect(svm, dvm)
                pltpu.async_copy(hs_hbm.at[svm], rows_v, sem).wait()
                pltpu.sync_copy(rows_v, sh.at[dvm], add=True)
                return 0

            lax.fori_loop(0, n1, body1, 0)

        @pl.when(c == 1)
        def _():
            def body3(t, _):
                base = (s * n3 + t) * CH
                pltpu.sync_copy(s3_hbm.at[pl.ds(base, CH)], svm)
                pltpu.sync_copy(d3_hbm.at[pl.ds(base, CH)], dvm)
                pltpu.async_copy(hs_hbm.at[svm], rows_v, sem).wait()
                pltpu.sync_copy(rows_v, sh.at[dvm], add=True)
                return 0

            lax.fori_loop(0, n3, body3, 0)

        plsc.subcore_barrier()
        rows = Nd // NS
        pltpu.sync_copy(sh.at[pl.ds(stripe, rows)],
                        o_hbm.at[c, pl.ds(stripe, rows)])

    return k(hspack, s1p, d1p, s3p, d3p, zeros128)


def kernel(x, edge_index, W1, b1, W2, b2, Wl, bl):
    src0 = edge_index[0].astype(jnp.int32)
    dst0 = edge_index[1].astype(jnp.int32)
    x_pad = jnp.pad(x, ((0, Nd - N), (0, 0)))

    loops = jnp.arange(N, dtype=jnp.int32)
    a_raw = jnp.zeros((Nd, Nd), jnp.bfloat16).at[dst0, src0].set(1.0)
    keep = 1.0 - (jnp.diagonal(a_raw)[:N] != 0).astype(jnp.float32)
    keep = jnp.pad(keep, (0, Nd - N))[:, None]
    atl = a_raw.at[loops, loops].set(1.0)

    s1p = jnp.concatenate([src0, jnp.full((E1P - E,), N, jnp.int32)])
    d1p = jnp.concatenate([dst0, jnp.full((E1P - E,), N, jnp.int32)])
    ones128 = jnp.ones((CH, 128), jnp.float32)
    zeros128 = jnp.zeros((CH, 128), jnp.float32)

    xn = _xn(x_pad)
    nbr = _knn(xn, xn.T)
    knn_dst = nbr[:N, :5].reshape(-1)
    knn_src = jnp.repeat(jnp.arange(N, dtype=jnp.int32), 5)
    s3p = jnp.concatenate([knn_src, jnp.full((E3P - 5 * N,), N, jnp.int32)])
    d3p = jnp.concatenate([knn_dst, jnp.full((E3P - 5 * N,), N, jnp.int32)])

    m2t, deg2 = _m2(atl)

    degp = _sc_deg(s1p, d1p, d3p, ones128, zeros128)
    deg1 = degp[0, :, :1]
    deg3 = degp[1, :, :1]

    b1r = b1[None, :]
    b2r = b2[None, :]
    blr = bl[None, :]

    h1, hs1 = _head1(x_pad, W1, deg1, deg3)
    op1 = _sc_layer(hs1, s1p, d1p, s3p, d3p, zeros128)
    ob1 = _dgcn(m2t, h1, deg2)

    h2, hs2 = _combine(op1[0, :, :64], op1[1, :, 64:], ob1, h1,
                       keep, deg1, deg3, W2, b1r)
    op2 = _sc_layer(hs2, s1p, d1p, s3p, d3p, zeros128)
    ob2 = _dgcn(m2t, h2, deg2)

    out = _final(op2[0, :, :64], op2[1, :, 64:], ob2, h2, keep,
                 deg1, deg3, Wl, b2r, blr)
    return out[:N]
